# baseline (speedup 1.0000x reference)
import jax
import jax.numpy as jnp
from jax import lax
from jax.experimental import pallas as pl
from jax.experimental.pallas import tpu as pltpu
from jax.experimental.pallas import tpu_sc as plsc

VOCAB = 1000000
DIM = 64
P = 4096
L = 50
LU = 200
NW = 32
GPW = P // NW
OPW = GPW * L
LANES = 16
INV = 1.0 / (float(L) * float(P))
SWEEP_BLK = 65536
NODE_WAY = 20


NODE_STEPS = LU // NODE_WAY
SWEEP_STEPS = (VOCAB + SWEEP_BLK - 1) // SWEEP_BLK


def _nodesweep_body(idx_ref, *refs):
    tbl_refs = refs[:NODE_WAY]
    sweep_ref, s_ref, node_scr = refs[NODE_WAY], refs[NODE_WAY + 1], refs[NODE_WAY + 2]
    i = pl.program_id(0)

    @pl.when(i == 0)
    def _():
        node_scr[...] = jnp.zeros_like(node_scr)

    @pl.when(i < NODE_STEPS)
    def _():
        acc = jnp.zeros((DIM, 1), jnp.float32)
        lanes = jax.lax.broadcasted_iota(jnp.int32, (DIM, 128), 1)
        for k in range(NODE_WAY):
            lane = idx_ref[i * NODE_WAY + k] % 128
            col = jnp.sum(jnp.where(lanes == lane, tbl_refs[k][...], 0.0),
                          axis=1, keepdims=True)
            acc = acc + col
        node_scr[...] += acc

    @pl.when(i == NODE_STEPS - 1)
    def _():
        node_scr[...] = node_scr[...] / float(LU)

    @pl.when(i >= NODE_STEPS)
    def _():
        s_ref[...] = jnp.sum(sweep_ref[...] * node_scr[...], axis=0)


def _nodesweep_call(pos_u, table_t):
    def mk_spec(k):
        def imap(i, idx_ref, k=k):
            ii = jnp.minimum(i, NODE_STEPS - 1)
            return (0, idx_ref[ii * NODE_WAY + k] // 128)
        return pl.BlockSpec((DIM, 128), imap)

    sweep_spec = pl.BlockSpec(
        (DIM, SWEEP_BLK), lambda i, idx_ref: (0, jnp.maximum(i - NODE_STEPS, 0)))

    return pl.pallas_call(
        _nodesweep_body,
        compiler_params=pltpu.CompilerParams(vmem_limit_bytes=56 * 1024 * 1024),
        grid_spec=pltpu.PrefetchScalarGridSpec(
            num_scalar_prefetch=1,
            grid=(NODE_STEPS + SWEEP_STEPS,),
            in_specs=[mk_spec(k) for k in range(NODE_WAY)] + [sweep_spec],
            out_specs=pl.BlockSpec(
                (SWEEP_BLK,), lambda i, idx_ref: (jnp.maximum(i - NODE_STEPS, 0),)),
            scratch_shapes=[pltpu.VMEM((DIM, 1), jnp.float32)],
        ),
        out_shape=jax.ShapeDtypeStruct((VOCAB,), jnp.float32),
    )(pos_u, *([table_t] * (NODE_WAY + 1)))


def _gather_body(s_hbm, negv_hbm, posv_hbm, exps_hbm, posp_hbm,
                 idxn_ref, idxp_ref, sbufn_ref, sbufp_ref, out_ref,
                 semn, semp, semi):
    core = lax.axis_index("core")
    sub = lax.axis_index("subcore")
    wid = sub * 2 + core
    col0 = wid * GPW

    cpn = pltpu.async_copy(negv_hbm.at[:, pl.ds(col0, GPW)], idxn_ref, semi)
    cpp = pltpu.async_copy(posv_hbm.at[:, pl.ds(col0, GPW)], idxp_ref, semi)
    cpn.wait()

    @pl.loop(0, L)
    def _fire_n(r):
        pltpu.async_copy(s_hbm.at[idxn_ref.at[r]], sbufn_ref.at[r], semn)

    cpp.wait()

    @pl.loop(0, L)
    def _fire_p(r):
        pltpu.async_copy(s_hbm.at[idxp_ref.at[r]], sbufp_ref.at[r], semp)

    @pl.loop(0, L)
    def _drain_n(r):
        pltpu.make_async_copy(
            s_hbm.at[idxn_ref.at[r]], sbufn_ref.at[r], semn).wait()

    expacc = jnp.zeros((LANES,), jnp.float32)
    for lc in range(GPW // LANES):

        def jbody(r, acc):
            return acc + sbufn_ref[r, pl.ds(lc * LANES, LANES)]

        gsum = lax.fori_loop(0, L, jbody, jnp.zeros((LANES,), jnp.float32))
        nr = gsum * INV
        expacc = expacc + jnp.where(nr > 0.0, jnp.exp(nr), 0.0)

    out_ref[0, :] = expacc
    pltpu.sync_copy(out_ref, exps_hbm.at[pl.ds(wid, 1)])

    @pl.loop(0, L)
    def _drain_p(r):
        pltpu.make_async_copy(
            s_hbm.at[idxp_ref.at[r]], sbufp_ref.at[r], semp).wait()

    def rbody(r, carry):
        acc = carry
        for lc in range(GPW // LANES):
            acc = acc + sbufp_ref[r, pl.ds(lc * LANES, LANES)]
        return acc

    posacc = lax.fori_loop(0, L, rbody, jnp.zeros((LANES,), jnp.float32))
    out_ref[0, :] = posacc
    pltpu.sync_copy(out_ref, posp_hbm.at[pl.ds(wid, 1)])


def _gather_call(s, negv_t, posv_t):
    mesh = plsc.VectorSubcoreMesh(core_axis_name="core", subcore_axis_name="subcore")
    f = pl.kernel(
        _gather_body,
        compiler_params=pltpu.CompilerParams(
            use_tc_tiling_on_sc=True, needs_layout_passes=False),
        out_type=[
            jax.ShapeDtypeStruct((NW, LANES), jnp.float32),
            jax.ShapeDtypeStruct((NW, LANES), jnp.float32),
        ],
        mesh=mesh,
        scratch_types=[
            pltpu.VMEM((L, GPW), jnp.int32),
            pltpu.VMEM((L, GPW), jnp.int32),
            pltpu.VMEM((L, GPW), jnp.float32),
            pltpu.VMEM((L, GPW), jnp.float32),
            pltpu.VMEM((1, LANES), jnp.float32),
            pltpu.SemaphoreType.DMA,
            pltpu.SemaphoreType.DMA,
            pltpu.SemaphoreType.DMA,
        ],
    )
    return f(s, negv_t, posv_t)


def kernel(pos_u, pos_v, neg_v, table):
    table_t = table.T
    posu = pos_u.astype(jnp.int32)
    negv_t = neg_v.astype(jnp.int32).T
    posv_t = pos_v.astype(jnp.int32).T

    s = _nodesweep_call(posu, table_t)
    exps, posp = _gather_call(s, negv_t, posv_t)

    neg_s = jnp.sum(exps)
    pos_loss = jnp.sum(posp) * INV
    return jnp.log(1.0 + neg_s) - pos_loss


# speedup vs baseline: 1.0551x; 1.0551x over previous
import jax
import jax.numpy as jnp
from jax import lax
from jax.experimental import pallas as pl
from jax.experimental.pallas import tpu as pltpu
from jax.experimental.pallas import tpu_sc as plsc

VOCAB = 1000000
VOCAB_PAD = 1048576
DIM = 64
P = 4096
L = 50
LU = 200
NW = 32
GPW = P // NW
OPW = GPW * L
LANES = 16
INV = 1.0 / (float(L) * float(P))
SWEEP_BLK = 65536
NODE_WAY = 20


NODE_STEPS = LU // NODE_WAY
SWEEP_STEPS = VOCAB_PAD // SWEEP_BLK


def _nodesweep_body(idx_ref, *refs):
    tbl_refs = refs[:NODE_WAY]
    sweep_ref, s_ref, node_scr = refs[NODE_WAY], refs[NODE_WAY + 1], refs[NODE_WAY + 2]
    i = pl.program_id(0)

    @pl.when(i == 0)
    def _():
        node_scr[...] = jnp.zeros_like(node_scr)

    @pl.when(i < NODE_STEPS)
    def _():
        acc = jnp.zeros((DIM, 1), jnp.float32)
        lanes = jax.lax.broadcasted_iota(jnp.int32, (DIM, 128), 1)
        for k in range(NODE_WAY):
            lane = idx_ref[i * NODE_WAY + k] % 128
            col = jnp.sum(jnp.where(lanes == lane, tbl_refs[k][...], 0.0),
                          axis=1, keepdims=True)
            acc = acc + col
        node_scr[...] += acc

    @pl.when(i == NODE_STEPS - 1)
    def _():
        node_scr[...] = node_scr[...] / float(LU)

    @pl.when(i >= NODE_STEPS)
    def _():
        s_ref[...] = jnp.sum(sweep_ref[...] * node_scr[...], axis=0)


def _nodesweep_call(pos_u, table_t):
    def mk_spec(k):
        def imap(i, idx_ref, k=k):
            ii = jnp.minimum(i, NODE_STEPS - 1)
            return (0, idx_ref[ii * NODE_WAY + k] // 128)
        return pl.BlockSpec((DIM, 128), imap)

    sweep_spec = pl.BlockSpec(
        (DIM, SWEEP_BLK), lambda i, idx_ref: (0, jnp.maximum(i - NODE_STEPS, 0)))

    return pl.pallas_call(
        _nodesweep_body,
        compiler_params=pltpu.CompilerParams(vmem_limit_bytes=56 * 1024 * 1024),
        grid_spec=pltpu.PrefetchScalarGridSpec(
            num_scalar_prefetch=1,
            grid=(NODE_STEPS + SWEEP_STEPS,),
            in_specs=[mk_spec(k) for k in range(NODE_WAY)] + [sweep_spec],
            out_specs=pl.BlockSpec(
                (SWEEP_BLK,), lambda i, idx_ref: (jnp.maximum(i - NODE_STEPS, 0),)),
            scratch_shapes=[pltpu.VMEM((DIM, 1), jnp.float32)],
        ),
        out_shape=jax.ShapeDtypeStruct((VOCAB_PAD,), jnp.float32),
    )(pos_u, *([table_t] * (NODE_WAY + 1)))


FILL = 65536


def _gather_body(s_hbm, negv_hbm, posv_hbm, exps_hbm, posp_hbm,
                 idxn_ref, idxp_ref, sbufn_ref, sbufp_ref, out_ref, s_shr,
                 semn, semp, semi):
    core = lax.axis_index("core")
    sub = lax.axis_index("subcore")
    wid = sub * 2 + core
    col0 = wid * GPW

    cpn = pltpu.async_copy(negv_hbm.at[:, pl.ds(col0, GPW)], idxn_ref, semi)
    cpp = pltpu.async_copy(posv_hbm.at[:, pl.ds(col0, GPW)], idxp_ref, semi)

    fb = sub * FILL
    pltpu.sync_copy(s_hbm.at[pl.ds(fb, FILL)], s_shr.at[pl.ds(fb, FILL)])

    cpn.wait()
    cpp.wait()
    plsc.subcore_barrier()

    @pl.loop(0, L)
    def _fire_n(r):
        pltpu.async_copy(s_shr.at[idxn_ref.at[r]], sbufn_ref.at[r], semn)

    @pl.loop(0, L)
    def _fire_p(r):
        pltpu.async_copy(s_shr.at[idxp_ref.at[r]], sbufp_ref.at[r], semp)

    @pl.loop(0, L)
    def _drain_n(r):
        pltpu.make_async_copy(
            s_shr.at[idxn_ref.at[r]], sbufn_ref.at[r], semn).wait()

    expacc = jnp.zeros((LANES,), jnp.float32)
    for lc in range(GPW // LANES):

        def jbody(r, acc):
            return acc + sbufn_ref[r, pl.ds(lc * LANES, LANES)]

        gsum = lax.fori_loop(0, L, jbody, jnp.zeros((LANES,), jnp.float32))
        nr = gsum * INV
        expacc = expacc + jnp.where(nr > 0.0, jnp.exp(nr), 0.0)

    out_ref[0, :] = expacc
    pltpu.sync_copy(out_ref, exps_hbm.at[pl.ds(wid, 1)])

    @pl.loop(0, L)
    def _drain_p(r):
        pltpu.make_async_copy(
            s_shr.at[idxp_ref.at[r]], sbufp_ref.at[r], semp).wait()

    def rbody(r, carry):
        acc = carry
        for lc in range(GPW // LANES):
            acc = acc + sbufp_ref[r, pl.ds(lc * LANES, LANES)]
        return acc

    posacc = lax.fori_loop(0, L, rbody, jnp.zeros((LANES,), jnp.float32))
    out_ref[0, :] = posacc
    pltpu.sync_copy(out_ref, posp_hbm.at[pl.ds(wid, 1)])


def _gather_call(s, negv_t, posv_t):
    mesh = plsc.VectorSubcoreMesh(core_axis_name="core", subcore_axis_name="subcore")
    f = pl.kernel(
        _gather_body,
        compiler_params=pltpu.CompilerParams(
            use_tc_tiling_on_sc=True, needs_layout_passes=False),
        out_type=[
            jax.ShapeDtypeStruct((NW, LANES), jnp.float32),
            jax.ShapeDtypeStruct((NW, LANES), jnp.float32),
        ],
        mesh=mesh,
        scratch_types=[
            pltpu.VMEM((L, GPW), jnp.int32),
            pltpu.VMEM((L, GPW), jnp.int32),
            pltpu.VMEM((L, GPW), jnp.float32),
            pltpu.VMEM((L, GPW), jnp.float32),
            pltpu.VMEM((1, LANES), jnp.float32),
            pltpu.VMEM_SHARED((VOCAB_PAD,), jnp.float32),
            pltpu.SemaphoreType.DMA,
            pltpu.SemaphoreType.DMA,
            pltpu.SemaphoreType.DMA,
        ],
    )
    return f(s, negv_t, posv_t)


def kernel(pos_u, pos_v, neg_v, table):
    table_t = table.T
    posu = pos_u.astype(jnp.int32)
    negv_t = neg_v.astype(jnp.int32).T
    posv_t = pos_v.astype(jnp.int32).T

    s = _nodesweep_call(posu, table_t)
    exps, posp = _gather_call(s, negv_t, posv_t)

    neg_s = jnp.sum(exps)
    pos_loss = jnp.sum(posp) * INV
    return jnp.log(1.0 + neg_s) - pos_loss
